# Initial kernel scaffold; baseline (speedup 1.0000x reference)
import jax
import jax.numpy as jnp
from jax.experimental import pallas as pl


def kernel(site_emb_w, site_emb_b, bond_emb_w, bond_emb_b, conv_wsig, conv_bsig, conv_wsoft, conv_bsoft, fc_w1, fc_b1, fc_w2, fc_b2, fc_w3, fc_b3, sites_raw, bonds_raw, idx1, idx2):
    raise NotImplementedError("write your pallas kernel here")


# speedup vs baseline: 3.4321x; 3.4321x over previous
import functools

import jax
import jax.numpy as jnp
from jax.experimental import pallas as pl
from jax.experimental.pallas import tpu as pltpu

_SUB = 16


def _trunk_kernel(gcat_ref, scat_ref, sraw_ref, draw_ref, mu_ref, sew_ref,
                  seb_ref, u0_ref, c0_ref, w12_ref, abond_ref, cbond_ref,
                  fw1_ref, fb1_ref, fw2_ref, fb2_ref, fw3t_ref, fb3_ref,
                  out_ref, *, n_layers, n_sites, n_edges, sub):
    S = sew_ref.shape[1]
    S2 = 2 * S
    N, E = n_sites, n_edges
    bf16, f32 = jnp.bfloat16, jnp.float32

    x = sraw_ref[0]
    s = x * sew_ref[...] + seb_ref[...]

    d = draw_ref[0]
    gb = jnp.exp(-jnp.square(d - mu_ref[...]))
    zb = jnp.dot(gb.astype(bf16), abond_ref[...],
                 preferred_element_type=f32) + cbond_ref[...]

    gcat = gcat_ref[...]
    scat = scat_ref[...]

    for l in range(n_layers):
        if l == 0:
            proj = (x * u0_ref[...] + c0_ref[...]).astype(bf16)
        else:
            proj = jnp.dot(s.astype(bf16), w12_ref[l],
                           preferred_element_type=f32).astype(bf16)
        p = jnp.concatenate(
            [jnp.concatenate([proj[b * N:(b + 1) * N, :S2],
                              proj[b * N:(b + 1) * N, S2:]], axis=0)
             for b in range(sub)], axis=1)
        z_all = jnp.dot(gcat, p, preferred_element_type=f32)
        zr = jnp.concatenate([z_all[:, b * S2:(b + 1) * S2]
                              for b in range(sub)], axis=0)
        z = zr + zb[:, l * S2:(l + 1) * S2]
        sig = pl.reciprocal(1.0 + jnp.exp(-z), approx=False)
        v = sig[:, :S] * jnp.maximum(z[:, S:], 0.0)
        vcat = jnp.concatenate([v[b * E:(b + 1) * E]
                                for b in range(sub)], axis=1).astype(bf16)
        delta = jnp.dot(scat, vcat, preferred_element_type=f32)
        s = s + jnp.concatenate([delta[:, b * S:(b + 1) * S]
                                 for b in range(sub)], axis=0)
    pooled = jnp.mean(s.reshape(sub, N, S), axis=1)
    h = jnp.maximum(jnp.dot(pooled, fw1_ref[...],
                            preferred_element_type=f32) + fb1_ref[...], 0.0)
    h = jnp.maximum(jnp.dot(h, fw2_ref[...],
                            preferred_element_type=f32) + fb2_ref[...], 0.0)
    out_ref[0] = (jnp.sum(h * fw3t_ref[...], axis=1, keepdims=True)
                  + fb3_ref[...])


def kernel(site_emb_w, site_emb_b, bond_emb_w, bond_emb_b,
           conv_wsig, conv_bsig, conv_wsoft, conv_bsoft,
           fc_w1, fc_b1, fc_w2, fc_b2, fc_w3, fc_b3,
           sites_raw, bonds_raw, idx1, idx2):
    f32, bf16 = jnp.float32, jnp.bfloat16
    B, N, _ = sites_raw.shape
    E = bonds_raw.shape[1]
    S = site_emb_w.shape[1]
    C = bond_emb_w.shape[0]
    Bn = bond_emb_w.shape[1]
    L = conv_wsig.shape[0]
    sub = _SUB if B % _SUB == 0 else (8 if B % 8 == 0 else 1)
    G = B // sub

    w_f = jnp.concatenate([conv_wsig, conv_wsoft], axis=-1).astype(f32)
    b_f = jnp.concatenate([conv_bsig, conv_bsoft], axis=-1).astype(f32)
    w12 = jnp.concatenate([w_f[:, :S, :], w_f[:, S:2 * S, :]], axis=2)
    w_bond = jnp.transpose(w_f[:, 2 * S:, :], (1, 0, 2)).reshape(Bn, L * 2 * S)
    a_bond = (bond_emb_w.astype(f32) @ w_bond).astype(bf16)
    c_bond = (bond_emb_b.astype(f32) @ w_bond
              + b_f.reshape(L * 2 * S))[None]

    u0 = site_emb_w.astype(f32) @ w12[0]
    c0 = site_emb_b[None].astype(f32) @ w12[0]

    mu = jnp.linspace(0.0, 10.0, C, dtype=f32)[None]

    oh1 = jax.nn.one_hot(idx1, N, dtype=f32)
    oh2 = jax.nn.one_hot(idx2, N, dtype=f32)
    gcat = jnp.concatenate([oh1, oh2], axis=1).astype(bf16)
    scat = oh1.T.astype(bf16)

    sraw = sites_raw.reshape(G, sub * N, 1).astype(f32)
    draw = bonds_raw.reshape(G, sub * E, 1).astype(f32)

    h1, h2 = fc_w1.shape[1], fc_w2.shape[1]
    kern = functools.partial(_trunk_kernel, n_layers=L, n_sites=N,
                             n_edges=E, sub=sub)
    out = pl.pallas_call(
        kern,
        out_shape=jax.ShapeDtypeStruct((G, sub, 1), f32),
        grid=(G,),
        in_specs=[
            pl.BlockSpec((E, 2 * N), lambda g: (0, 0)),
            pl.BlockSpec((N, E), lambda g: (0, 0)),
            pl.BlockSpec((1, sub * N, 1), lambda g: (g, 0, 0)),
            pl.BlockSpec((1, sub * E, 1), lambda g: (g, 0, 0)),
            pl.BlockSpec((1, C), lambda g: (0, 0)),
            pl.BlockSpec((1, S), lambda g: (0, 0)),
            pl.BlockSpec((1, S), lambda g: (0, 0)),
            pl.BlockSpec((1, 4 * S), lambda g: (0, 0)),
            pl.BlockSpec((1, 4 * S), lambda g: (0, 0)),
            pl.BlockSpec((L, S, 4 * S), lambda g: (0, 0, 0)),
            pl.BlockSpec((C, L * 2 * S), lambda g: (0, 0)),
            pl.BlockSpec((1, L * 2 * S), lambda g: (0, 0)),
            pl.BlockSpec((S, h1), lambda g: (0, 0)),
            pl.BlockSpec((1, h1), lambda g: (0, 0)),
            pl.BlockSpec((h1, h2), lambda g: (0, 0)),
            pl.BlockSpec((1, h2), lambda g: (0, 0)),
            pl.BlockSpec((1, h2), lambda g: (0, 0)),
            pl.BlockSpec((1, 1), lambda g: (0, 0)),
        ],
        out_specs=pl.BlockSpec((1, sub, 1), lambda g: (g, 0, 0)),
        compiler_params=pltpu.CompilerParams(
            dimension_semantics=("parallel",),
            vmem_limit_bytes=64 * 1024 * 1024),
    )(gcat, scat, sraw, draw, mu,
      site_emb_w.astype(f32), site_emb_b[None].astype(f32),
      u0, c0, w12.astype(bf16), a_bond, c_bond,
      fc_w1.astype(f32), fc_b1[None].astype(f32),
      fc_w2.astype(f32), fc_b2[None].astype(f32),
      fc_w3.reshape(1, h2).astype(f32), fc_b3.reshape(1, 1).astype(f32))
    return out.reshape(B, 1)
